# baseline (speedup 1.0000x reference)
import functools

import jax
import jax.numpy as jnp
from jax import lax
from jax.experimental import pallas as pl
from jax.experimental.pallas import tpu as pltpu
from jax.experimental.pallas import tpu_sc as plsc

_N_GARMENT = 21
_N_INDEX = 10

_B = 16384
_V = 1000001
_VPAD = 1000064
_D = 64
_NW = 32
_NL = 16
_SHARD = 32768
_WIN = 512
_NWIN = _SHARD // _WIN
_SEG = 17408
_OCAP = 17408
_JUNK = _B


def _iota16():
    return lax.broadcasted_iota(jnp.int32, (_NL,), 0)


def _vfull(x):
    return jnp.full((_NL,), x, jnp.int32)


def _scalar_store(ref, index, value):
    plsc.store_scatter(ref, [jnp.full((_NL,), index, jnp.int32)],
                       jnp.full((_NL,), value, jnp.int32))


def _sc_route(idx):
    mesh = plsc.VectorSubcoreMesh(core_axis_name="c", subcore_axis_name="s")

    @functools.partial(
        pl.kernel,
        mesh=mesh,
        compiler_params=pltpu.CompilerParams(use_tc_tiling_on_sc=False,
                                             needs_layout_passes=False),
        out_type=(
            jax.ShapeDtypeStruct((_NW * _SEG,), jnp.int32),
            jax.ShapeDtypeStruct((_NW * _SEG,), jnp.int32),
            jax.ShapeDtypeStruct((_NW * 128,), jnp.int32),
        ),
        scratch_types=[
            pltpu.VMEM((_B,), jnp.int32),
            pltpu.VMEM((_SEG + _NL,), jnp.int32),
            pltpu.VMEM((_SEG + _NL,), jnp.int32),
            pltpu.VMEM((_SEG + _NL,), jnp.int32),
            pltpu.VMEM((_SEG + _NL,), jnp.int32),
            pltpu.VMEM((128,), jnp.int32),
            pltpu.VMEM((_NL,), jnp.int32),
            pltpu.VMEM((16 * _NL,), jnp.int32),
            pltpu.VMEM_SHARED((16 * _NL,), jnp.int32),
        ],
    )
    def route_kernel(idx_hbm, ks_hbm, ps_hbm, meta_hbm, idx_v, ks_v, ps_v,
                     kso_v, pso_v, meta_v, tmp_v, all_v, spm):
        cid = lax.axis_index("c")
        sid = lax.axis_index("s")
        wid = sid * 2 + cid
        pltpu.sync_copy(idx_hbm, idx_v)
        iota = _iota16()

        def filt(v, off):
            kv = idx_v[pl.ds(v * _NL, _NL)]
            pv = _vfull(v * _NL) + iota
            m = (kv >> 15) == _vfull(wid)
            cs = plsc.cumsum(jnp.where(m, _vfull(1), _vfull(0)))
            tgt = jnp.where(m, _vfull(off - 1) + cs, _vfull(_SEG))
            plsc.store_scatter(ks_v, [tgt], kv)
            plsc.store_scatter(ps_v, [tgt], pv)
            return off + cs[_NL - 1]

        n = lax.fori_loop(0, _B // _NL, filt, 0)
        nv = (n + _NL - 1) // _NL

        def group(b, off2):
            _scalar_store(meta_v, b, off2)
            vbase = wid * _SHARD + b * _WIN

            def pick(j, o):
                e = _vfull(j * _NL) + iota
                kv = plsc.load_gather(ks_v, [e])
                pv = plsc.load_gather(ps_v, [e])
                m = (e < _vfull(n)) & ((kv >> 9) == _vfull(wid * _NWIN + b))
                cs = plsc.cumsum(jnp.where(m, _vfull(1), _vfull(0)))
                tgt = jnp.where(m, _vfull(o - 1) + cs, _vfull(_SEG))
                plsc.store_scatter(kso_v, [tgt], kv)
                plsc.store_scatter(pso_v, [tgt], pv)
                return o + cs[_NL - 1]

            end = lax.fori_loop(0, nv, pick, off2)
            plsc.store_scatter(kso_v, [_vfull(end) + iota], _vfull(vbase))
            plsc.store_scatter(pso_v, [_vfull(end) + iota], _vfull(_JUNK))
            return (end + _NL - 1) & ~(_NL - 1)

        np_ = lax.fori_loop(0, _NWIN, group, 0)
        _scalar_store(meta_v, 64, np_)

        out_len = (np_ + 63) & ~63
        _scalar_store(tmp_v, 0, out_len)
        pltpu.sync_copy(tmp_v, spm.at[pl.ds(sid * _NL, _NL)])
        plsc.subcore_barrier()
        pltpu.sync_copy(spm, all_v)
        lens = plsc.load_gather(all_v, [iota * _vfull(_NL)])
        excl = plsc.cumsum(lens) - lens
        plsc.store_scatter(tmp_v, [iota], excl)
        my_start = plsc.load_gather(tmp_v, [jnp.full((_NL,), sid, jnp.int32)])[0]
        out_start = cid * _OCAP + my_start
        _scalar_store(meta_v, 65, out_start)
        _scalar_store(meta_v, 66, out_len)

        pltpu.sync_copy(kso_v.at[pl.ds(0, _SEG)],
                        ks_hbm.at[pl.ds(pl.multiple_of(wid * _SEG, 128), _SEG)])
        pltpu.sync_copy(pso_v.at[pl.ds(0, _SEG)],
                        ps_hbm.at[pl.ds(pl.multiple_of(wid * _SEG, 128), _SEG)])
        pltpu.sync_copy(meta_v, meta_hbm.at[pl.ds(pl.multiple_of(wid * 128, 128), 128)])

    return route_kernel(idx)


def _sc_sweep(table_t, ks2, ps2, meta):
    mesh = plsc.VectorSubcoreMesh(core_axis_name="c", subcore_axis_name="s")

    @functools.partial(
        pl.kernel,
        mesh=mesh,
        compiler_params=pltpu.CompilerParams(needs_layout_passes=False),
        out_type=(
            jax.ShapeDtypeStruct((2 * _OCAP, _D), jnp.float32),
            jax.ShapeDtypeStruct((2 * _OCAP,), jnp.int32),
        ),
        scratch_types=[
            pltpu.VMEM((_D, _WIN), jnp.float32),
            pltpu.VMEM((_D, _WIN), jnp.float32),
            pltpu.VMEM((_SEG,), jnp.int32),
            pltpu.VMEM((_SEG,), jnp.int32),
            pltpu.VMEM((128,), jnp.int32),
            pltpu.VMEM((64, _D), jnp.float32),
            pltpu.VMEM((64,), jnp.int32),
            pltpu.SemaphoreType.DMA,
            pltpu.SemaphoreType.DMA,
        ],
    )
    def sweep_kernel(tab_hbm, ks_hbm, ps_hbm, meta_hbm, out_hbm, pos_hbm,
                     buf0, buf1, kso_v, pso_v, meta_v, st_v, pst_v,
                     sem0, sem1):
        cid = lax.axis_index("c")
        sid = lax.axis_index("s")
        wid = sid * 2 + cid
        iota = _iota16()
        pltpu.sync_copy(meta_hbm.at[pl.ds(pl.multiple_of(wid * 128, 128), 128)], meta_v)
        pltpu.sync_copy(ks_hbm.at[pl.ds(pl.multiple_of(wid * _SEG, 128), _SEG)], kso_v)
        pltpu.sync_copy(ps_hbm.at[pl.ds(pl.multiple_of(wid * _SEG, 128), _SEG)], pso_v)
        mv = meta_v[pl.ds(64, _NL)]
        out_start = pl.multiple_of(mv[1], 64)
        junkvec = jnp.full((_NL,), _JUNK, jnp.int32)

        def init_pst():
            for q in range(4):
                pst_v[pl.ds(q * _NL, _NL)] = junkvec

        init_pst()

        def vbase_of(w):
            return wid * _SHARD + w * _WIN

        def issue(w, buf, sem):
            vb = vbase_of(w)

            @pl.when(jnp.logical_and(w < _NWIN, vb + _WIN <= _VPAD))
            def _():
                pltpu.async_copy(tab_hbm.at[:, pl.ds(vb, _WIN)], buf, sem)

            @pl.when(jnp.logical_and(w < _NWIN,
                                     jnp.logical_and(vb < _VPAD,
                                                     vb + _WIN > _VPAD)))
            def _():
                pltpu.async_copy(tab_hbm.at[:, pl.ds(vb, 128)],
                                 buf.at[:, pl.ds(0, 128)], sem)

        def wait(w, buf, sem):
            vb = vbase_of(w)

            @pl.when(vb + _WIN <= _VPAD)
            def _():
                pltpu.make_async_copy(tab_hbm.at[:, pl.ds(0, _WIN)], buf,
                                      sem).wait()

            @pl.when(jnp.logical_and(vb < _VPAD, vb + _WIN > _VPAD))
            def _():
                pltpu.make_async_copy(tab_hbm.at[:, pl.ds(0, 128)],
                                      buf.at[:, pl.ds(0, 128)], sem).wait()

        fill_base = cid * _OCAP + sid * (_OCAP // 16)
        for f in range(_OCAP // 16 // 64):
            pltpu.sync_copy(
                pst_v,
                pos_hbm.at[pl.ds(pl.multiple_of(fill_base + f * 64, 64), 64)])
        plsc.subcore_barrier()

        issue(0, buf0, sem0)
        issue(1, buf1, sem1)

        def extract(w, buf, carry):
            slot, optr = carry
            vb = vbase_of(w)
            g0 = plsc.load_gather(meta_v, [jnp.full((_NL,), w, jnp.int32)])[0]
            g1 = plsc.load_gather(
                meta_v, [jnp.full((_NL,), w + 1, jnp.int32)])[0]

            def chunk(j, carry2):
                slot, optr = carry2
                e = _vfull(g0 + j * _NL) + iota
                kv = plsc.load_gather(kso_v, [e])
                pv = plsc.load_gather(pso_v, [e])
                colv = kv - _vfull(vb)

                def flush(slot, optr):
                    o = pl.multiple_of(optr, 64)
                    pltpu.sync_copy(st_v, out_hbm.at[pl.ds(o, 64)])
                    pltpu.sync_copy(pst_v, pos_hbm.at[pl.ds(o, 64)])
                    init_pst()
                    return 0, optr + 64

                for l in range(_NL):
                    col16 = _vfull(colv[l])
                    slot16 = _vfull(slot)
                    for q in range(4):
                        rows = _vfull(q * _NL) + iota
                        vals = plsc.load_gather(buf, [rows, col16])
                        plsc.store_scatter(st_v, [slot16, rows], vals)
                    plsc.store_scatter(pst_v, [slot16], _vfull(pv[l]))
                    slot = slot + 1
                    slot, optr = lax.cond(slot == 64, flush,
                                          lambda s, o: (s, o), slot, optr)
                return slot, optr

            nvw = (g1 - g0) // _NL
            return lax.fori_loop(0, nvw, chunk, (slot, optr))

        def pair(g, carry):
            for p, (buf, sem) in enumerate(((buf0, sem0), (buf1, sem1))):
                w = 2 * g + p
                wait(w, buf, sem)
                carry = extract(w, buf, carry)
                issue(w + 2, buf, sem)
            return carry

        slot, optr = lax.fori_loop(0, _NWIN // 2, pair, (0, out_start))

        @pl.when(slot > 0)
        def _():
            o = pl.multiple_of(optr, 64)
            pltpu.sync_copy(st_v, out_hbm.at[pl.ds(o, 64)])
            pltpu.sync_copy(pst_v, pos_hbm.at[pl.ds(o, 64)])

    return sweep_kernel(table_t, ks2, ps2, meta)


def _sc_unperm(out1, pos1, meta):
    mesh = plsc.VectorSubcoreMesh(core_axis_name="c", subcore_axis_name="s")
    tot = 2 * _OCAP

    @functools.partial(
        pl.kernel,
        mesh=mesh,
        compiler_params=pltpu.CompilerParams(use_tc_tiling_on_sc=False,
                                             needs_layout_passes=False),
        out_type=jax.ShapeDtypeStruct((_B, _D), jnp.float32),
        scratch_types=[
            pltpu.VMEM((tot,), jnp.int32),
            pltpu.VMEM((5, 128), jnp.int32),
            pltpu.VMEM((512, _D), jnp.float32),
            pltpu.SemaphoreType.DMA,
        ],
    )
    def unperm_kernel(rows_hbm, pos_hbm, meta_hbm, out_hbm, pos_v, inv_v,
                      rv, sem):
        del meta_hbm
        cid = lax.axis_index("c")
        sid = lax.axis_index("s")
        wid = sid * 2 + cid
        base = wid * 512
        iota = _iota16()
        pltpu.sync_copy(pos_hbm, pos_v)

        def scan(v, carry):
            pv = pos_v[pl.ds(v * _NL, _NL)]
            loc = _vfull(v * _NL) + iota
            rel = pv - _vfull(base)
            m = (rel >= _vfull(0)) & (rel < _vfull(512))
            flat = jnp.where(m, rel, _vfull(512))
            plsc.store_scatter(inv_v, [flat >> 7, flat & _vfull(127)], loc)
            return carry

        lax.fori_loop(0, tot // _NL, scan, 0)
        for q in range(4):
            pltpu.async_copy(rows_hbm.at[inv_v.at[q]],
                             rv.at[pl.ds(q * 128, 128)], sem)
        pltpu.make_async_copy(rows_hbm.at[pl.ds(0, 512)], rv, sem).wait()
        pltpu.sync_copy(rv, out_hbm.at[pl.ds(pl.multiple_of(base, 128), 512)])

    return unperm_kernel(out1, pos1, meta)


def _mlp_t_body(embt_ref, g_ref, i_ref, w1t_ref, b1_ref, w2t_ref, b2_ref,
                o_ref):
    blk = embt_ref.shape[1]
    e = embt_ref[...]
    g = g_ref[...]
    i = i_ref[...]
    row = lax.broadcasted_iota(jnp.int32, (_D, blk), 0)
    oht = ((row == g) | (row == i + _N_GARMENT)).astype(jnp.float32)
    cct = jnp.concatenate([e, oht], axis=0)
    ht = jnp.dot(w1t_ref[...], cct, preferred_element_type=jnp.float32)
    ht = jnp.maximum(ht + b1_ref[...], 0.0)
    ot = jnp.dot(w2t_ref[...], ht, preferred_element_type=jnp.float32)
    o_ref[...] = ot + b2_ref[...]


def _mlp_t(emb_t, gid, iid, w1pt, b1c, w2t, b2c):
    d, b = emb_t.shape
    blk = 2048
    grid = b // blk
    return pl.pallas_call(
        _mlp_t_body,
        grid=(grid,),
        in_specs=[
            pl.BlockSpec((d, blk), lambda i: (0, i)),
            pl.BlockSpec((1, blk), lambda i: (0, i)),
            pl.BlockSpec((1, blk), lambda i: (0, i)),
            pl.BlockSpec((d, 128), lambda i: (0, 0)),
            pl.BlockSpec((d, 1), lambda i: (0, 0)),
            pl.BlockSpec((d, d), lambda i: (0, 0)),
            pl.BlockSpec((d, 1), lambda i: (0, 0)),
        ],
        out_specs=pl.BlockSpec((d, blk), lambda i: (0, i)),
        out_shape=jax.ShapeDtypeStruct((d, b), jnp.float32),
    )(emb_t, gid.reshape(1, b), iid.reshape(1, b), w1pt, b1c, w2t, b2c)


def kernel(article_id, garment_group_name, index_group_name, emb_table,
           W1, b1, W2, b2):
    ks2, ps2, meta = _sc_route(article_id)
    out1, pos1 = _sc_sweep(emb_table.T, ks2, ps2, meta)
    emb = _sc_unperm(out1, pos1, meta)
    d = W1.shape[1]
    w1pt = jnp.zeros((d, 128), W1.dtype).at[:, : W1.shape[0]].set(W1.T)
    out_t = _mlp_t(emb.T, garment_group_name, index_group_name, w1pt,
                   b1.reshape(d, 1), W2.T, b2.reshape(d, 1))
    return out_t.T


# speedup vs baseline: 1.0385x; 1.0385x over previous
import functools

import jax
import jax.numpy as jnp
from jax import lax
from jax.experimental import pallas as pl
from jax.experimental.pallas import tpu as pltpu
from jax.experimental.pallas import tpu_sc as plsc

_N_GARMENT = 21
_N_INDEX = 10

_B = 16384
_V = 1000001
_VPAD = 1000064
_D = 64
_NW = 32
_NL = 16
_SHARD = 32768
_WIN = 512
_NWIN = _SHARD // _WIN
_SEG = 17408
_OCAP = 17408
_JUNK = _B


def _iota16():
    return lax.broadcasted_iota(jnp.int32, (_NL,), 0)


def _vfull(x):
    return jnp.full((_NL,), x, jnp.int32)


def _scalar_store(ref, index, value):
    plsc.store_scatter(ref, [jnp.full((_NL,), index, jnp.int32)],
                       jnp.full((_NL,), value, jnp.int32))


def _sc_gather_packed(table_t, idx):
    mesh = plsc.VectorSubcoreMesh(core_axis_name="c", subcore_axis_name="s")

    @functools.partial(
        pl.kernel,
        mesh=mesh,
        compiler_params=pltpu.CompilerParams(needs_layout_passes=False),
        out_type=(
            jax.ShapeDtypeStruct((2 * _OCAP, _D), jnp.float32),
            jax.ShapeDtypeStruct((2 * _OCAP,), jnp.int32),
        ),
        scratch_types=[
            pltpu.VMEM((_D, _WIN), jnp.float32),
            pltpu.VMEM((_D, _WIN), jnp.float32),
            pltpu.VMEM((_B,), jnp.int32),
            pltpu.VMEM((_SEG + _NL,), jnp.int32),
            pltpu.VMEM((_SEG + _NL,), jnp.int32),
            pltpu.VMEM((128,), jnp.int32),
            pltpu.VMEM((64, _D), jnp.float32),
            pltpu.VMEM((64,), jnp.int32),
            pltpu.VMEM((_NL,), jnp.int32),
            pltpu.VMEM((16 * _NL,), jnp.int32),
            pltpu.VMEM_SHARED((16 * _NL,), jnp.int32),
            pltpu.SemaphoreType.DMA,
            pltpu.SemaphoreType.DMA,
        ],
    )
    def gather_kernel(tab_hbm, idx_hbm, out_hbm, pos_hbm, buf0, buf1, idx_v,
                      pk_v, pko_v, meta_v, st_v, pst_v, tmp_v, all_v, spm,
                      sem0, sem1):
        cid = lax.axis_index("c")
        sid = lax.axis_index("s")
        wid = sid * 2 + cid
        iota = _iota16()
        junkvec = jnp.full((_NL,), _JUNK, jnp.int32)

        def init_pst():
            for q in range(4):
                pst_v[pl.ds(q * _NL, _NL)] = junkvec

        init_pst()

        def vbase_of(w):
            return wid * _SHARD + w * _WIN

        def issue(w, buf, sem):
            vb = vbase_of(w)

            @pl.when(jnp.logical_and(w < _NWIN, vb + _WIN <= _VPAD))
            def _():
                pltpu.async_copy(tab_hbm.at[:, pl.ds(vb, _WIN)], buf, sem)

            @pl.when(jnp.logical_and(w < _NWIN,
                                     jnp.logical_and(vb < _VPAD,
                                                     vb + _WIN > _VPAD)))
            def _():
                pltpu.async_copy(tab_hbm.at[:, pl.ds(vb, 128)],
                                 buf.at[:, pl.ds(0, 128)], sem)

        def wait(w, buf, sem):
            vb = vbase_of(w)

            @pl.when(vb + _WIN <= _VPAD)
            def _():
                pltpu.make_async_copy(tab_hbm.at[:, pl.ds(0, _WIN)], buf,
                                      sem).wait()

            @pl.when(jnp.logical_and(vb < _VPAD, vb + _WIN > _VPAD))
            def _():
                pltpu.make_async_copy(tab_hbm.at[:, pl.ds(0, 128)],
                                      buf.at[:, pl.ds(0, 128)], sem).wait()

        fill_base = cid * _OCAP + sid * (_OCAP // 16)
        for f in range(_OCAP // 16 // 64):
            pltpu.sync_copy(
                pst_v,
                pos_hbm.at[pl.ds(pl.multiple_of(fill_base + f * 64, 64), 64)])
        plsc.subcore_barrier()

        issue(0, buf0, sem0)
        issue(1, buf1, sem1)

        pltpu.sync_copy(idx_hbm, idx_v)

        def filt(v, off):
            kv = idx_v[pl.ds(pl.multiple_of(v * _NL, _NL), _NL)]
            pv = _vfull(v * _NL) + iota
            m = (kv >> 15) == _vfull(wid)
            packed = ((kv & _vfull(_SHARD - 1)) << 15) | pv
            cs = plsc.cumsum(jnp.where(m, _vfull(1), _vfull(0)))
            tgt = jnp.where(m, _vfull(off - 1) + cs, _vfull(_SEG))
            plsc.store_scatter(pk_v, [tgt], packed)
            return off + cs[_NL - 1]

        n = lax.fori_loop(0, _B // _NL, filt, 0)
        nv = (n + _NL - 1) // _NL

        def group(b, off2):
            _scalar_store(meta_v, b, off2)

            def pick(j, o):
                e = _vfull(j * _NL) + iota
                ev = plsc.load_gather(pk_v, [e])
                m = (e < _vfull(n)) & ((ev >> 24) == _vfull(b))
                cs = plsc.cumsum(jnp.where(m, _vfull(1), _vfull(0)))
                tgt = jnp.where(m, _vfull(o - 1) + cs, _vfull(_SEG))
                plsc.store_scatter(pko_v, [tgt], ev)
                return o + cs[_NL - 1]

            end = lax.fori_loop(0, nv, pick, off2)
            sent = ((b * _WIN) << 15) | _JUNK
            plsc.store_scatter(pko_v, [_vfull(end) + iota], _vfull(sent))
            return (end + _NL - 1) & ~(_NL - 1)

        np_ = lax.fori_loop(0, _NWIN, group, 0)
        _scalar_store(meta_v, 64, np_)

        out_len = (np_ + 63) & ~63
        _scalar_store(tmp_v, 0, out_len)
        pltpu.sync_copy(tmp_v,
                        spm.at[pl.ds(pl.multiple_of(sid * _NL, _NL), _NL)])
        plsc.subcore_barrier()
        pltpu.sync_copy(spm, all_v)
        lens = plsc.load_gather(all_v, [iota * _vfull(_NL)])
        excl = plsc.cumsum(lens) - lens
        plsc.store_scatter(tmp_v, [iota], excl)
        my_start = plsc.load_gather(tmp_v, [_vfull(sid)])[0]
        out_start = pl.multiple_of(cid * _OCAP + my_start, 64)

        def extract(w, buf, carry):
            slot, optr = carry
            g0 = plsc.load_gather(meta_v, [_vfull(w)])[0]
            g1 = plsc.load_gather(meta_v, [_vfull(w + 1)])[0]

            def chunk(j, carry2):
                slot, optr = carry2
                e = _vfull(g0 + j * _NL) + iota
                ev = plsc.load_gather(pko_v, [e])
                colv = (ev >> 15) - _vfull(w * _WIN)
                pvv = ev & _vfull(32767)

                def flush(slot, optr):
                    o = pl.multiple_of(optr, 64)
                    pltpu.sync_copy(st_v, out_hbm.at[pl.ds(o, 64)])
                    pltpu.sync_copy(pst_v, pos_hbm.at[pl.ds(o, 64)])
                    init_pst()
                    return 0, optr + 64

                for l in range(_NL):
                    col16 = _vfull(colv[l])
                    slot16 = _vfull(slot)
                    for q in range(4):
                        rows = _vfull(q * _NL) + iota
                        vals = plsc.load_gather(buf, [rows, col16])
                        plsc.store_scatter(st_v, [slot16, rows], vals)
                    plsc.store_scatter(pst_v, [slot16], _vfull(pvv[l]))
                    slot = slot + 1
                    slot, optr = lax.cond(slot == 64, flush,
                                          lambda s, o: (s, o), slot, optr)
                return slot, optr

            nvw = (g1 - g0) // _NL
            return lax.fori_loop(0, nvw, chunk, (slot, optr))

        def pair(g, carry):
            for p, (buf, sem) in enumerate(((buf0, sem0), (buf1, sem1))):
                w = 2 * g + p
                wait(w, buf, sem)
                carry = extract(w, buf, carry)
                issue(w + 2, buf, sem)
            return carry

        slot, optr = lax.fori_loop(0, _NWIN // 2, pair, (0, out_start))

        @pl.when(slot > 0)
        def _():
            o = pl.multiple_of(optr, 64)
            pltpu.sync_copy(st_v, out_hbm.at[pl.ds(o, 64)])
            pltpu.sync_copy(pst_v, pos_hbm.at[pl.ds(o, 64)])

    return gather_kernel(table_t, idx)


def _sc_unperm(out1, pos1):
    mesh = plsc.VectorSubcoreMesh(core_axis_name="c", subcore_axis_name="s")
    tot = 2 * _OCAP

    @functools.partial(
        pl.kernel,
        mesh=mesh,
        compiler_params=pltpu.CompilerParams(use_tc_tiling_on_sc=False,
                                             needs_layout_passes=False),
        out_type=jax.ShapeDtypeStruct((_B, _D), jnp.float32),
        scratch_types=[
            pltpu.VMEM((tot,), jnp.int32),
            pltpu.VMEM((5, 128), jnp.int32),
            pltpu.VMEM((512, _D), jnp.float32),
            pltpu.SemaphoreType.DMA,
        ],
    )
    def unperm_kernel(rows_hbm, pos_hbm, out_hbm, pos_v, inv_v,
                      rv, sem):
        cid = lax.axis_index("c")
        sid = lax.axis_index("s")
        wid = sid * 2 + cid
        base = wid * 512
        iota = _iota16()
        pltpu.sync_copy(pos_hbm, pos_v)

        def scan(v, carry):
            pv = pos_v[pl.ds(v * _NL, _NL)]
            loc = _vfull(v * _NL) + iota
            rel = pv - _vfull(base)
            m = (rel >= _vfull(0)) & (rel < _vfull(512))
            flat = jnp.where(m, rel, _vfull(512))
            plsc.store_scatter(inv_v, [flat >> 7, flat & _vfull(127)], loc)
            return carry

        lax.fori_loop(0, tot // _NL, scan, 0)
        for q in range(4):
            pltpu.async_copy(rows_hbm.at[inv_v.at[q]],
                             rv.at[pl.ds(q * 128, 128)], sem)
        pltpu.make_async_copy(rows_hbm.at[pl.ds(0, 512)], rv, sem).wait()
        pltpu.sync_copy(rv, out_hbm.at[pl.ds(pl.multiple_of(base, 128), 512)])

    return unperm_kernel(out1, pos1)


def _mlp_t_body(embt_ref, g_ref, i_ref, w1t_ref, b1_ref, w2t_ref, b2_ref,
                o_ref):
    blk = embt_ref.shape[1]
    e = embt_ref[...]
    g = g_ref[...]
    i = i_ref[...]
    row = lax.broadcasted_iota(jnp.int32, (_D, blk), 0)
    oht = ((row == g) | (row == i + _N_GARMENT)).astype(jnp.float32)
    cct = jnp.concatenate([e, oht], axis=0)
    ht = jnp.dot(w1t_ref[...], cct, preferred_element_type=jnp.float32)
    ht = jnp.maximum(ht + b1_ref[...], 0.0)
    ot = jnp.dot(w2t_ref[...], ht, preferred_element_type=jnp.float32)
    o_ref[...] = ot + b2_ref[...]


def _mlp_t(emb_t, gid, iid, w1pt, b1c, w2t, b2c):
    d, b = emb_t.shape
    blk = 2048
    grid = b // blk
    return pl.pallas_call(
        _mlp_t_body,
        grid=(grid,),
        in_specs=[
            pl.BlockSpec((d, blk), lambda i: (0, i)),
            pl.BlockSpec((1, blk), lambda i: (0, i)),
            pl.BlockSpec((1, blk), lambda i: (0, i)),
            pl.BlockSpec((d, 128), lambda i: (0, 0)),
            pl.BlockSpec((d, 1), lambda i: (0, 0)),
            pl.BlockSpec((d, d), lambda i: (0, 0)),
            pl.BlockSpec((d, 1), lambda i: (0, 0)),
        ],
        out_specs=pl.BlockSpec((d, blk), lambda i: (0, i)),
        out_shape=jax.ShapeDtypeStruct((d, b), jnp.float32),
    )(emb_t, gid.reshape(1, b), iid.reshape(1, b), w1pt, b1c, w2t, b2c)


def kernel(article_id, garment_group_name, index_group_name, emb_table,
           W1, b1, W2, b2):
    out1, pos1 = _sc_gather_packed(emb_table.T, article_id)
    emb = _sc_unperm(out1, pos1)
    d = W1.shape[1]
    w1pt = jnp.zeros((d, 128), W1.dtype).at[:, : W1.shape[0]].set(W1.T)
    out_t = _mlp_t(emb.T, garment_group_name, index_group_name, w1pt,
                   b1.reshape(d, 1), W2.T, b2.reshape(d, 1))
    return out_t.T


# speedup vs baseline: 1.2511x; 1.2048x over previous
import functools

import jax
import jax.numpy as jnp
from jax import lax
from jax.experimental import pallas as pl
from jax.experimental.pallas import tpu as pltpu
from jax.experimental.pallas import tpu_sc as plsc

_N_GARMENT = 21
_N_INDEX = 10

_B = 16384
_V = 1000001
_VPAD = 1000064
_D = 64
_NW = 32
_NL = 16
_SHARD = 32768
_WIN = 512
_NWIN = _SHARD // _WIN
_SEG = 17408
_OCAP = 17408
_JUNK = _B


def _iota16():
    return lax.broadcasted_iota(jnp.int32, (_NL,), 0)


def _vfull(x):
    return jnp.full((_NL,), x, jnp.int32)


def _scalar_store(ref, index, value):
    plsc.store_scatter(ref, [jnp.full((_NL,), index, jnp.int32)],
                       jnp.full((_NL,), value, jnp.int32))


def _sc_gather_packed(table_t, idx):
    mesh = plsc.VectorSubcoreMesh(core_axis_name="c", subcore_axis_name="s")

    @functools.partial(
        pl.kernel,
        mesh=mesh,
        compiler_params=pltpu.CompilerParams(needs_layout_passes=False),
        out_type=(
            jax.ShapeDtypeStruct((2 * _OCAP, _D), jnp.float32),
            jax.ShapeDtypeStruct((2 * _OCAP,), jnp.int32),
        ),
        scratch_types=[
            pltpu.VMEM((_D, _WIN), jnp.float32),
            pltpu.VMEM((_D, _WIN), jnp.float32),
            pltpu.VMEM((_B,), jnp.int32),
            pltpu.VMEM((_SEG + _NL,), jnp.int32),
            pltpu.VMEM((_SEG + _NL,), jnp.int32),
            pltpu.VMEM((128,), jnp.int32),
            pltpu.VMEM((64, _D), jnp.float32),
            pltpu.VMEM((64,), jnp.int32),
            pltpu.VMEM((_NL,), jnp.int32),
            pltpu.VMEM((16 * _NL,), jnp.int32),
            pltpu.VMEM_SHARED((16 * _NL,), jnp.int32),
            pltpu.SemaphoreType.DMA,
            pltpu.SemaphoreType.DMA,
        ],
    )
    def gather_kernel(tab_hbm, idx_hbm, out_hbm, pos_hbm, buf0, buf1, idx_v,
                      pk_v, pko_v, meta_v, st_v, pst_v, tmp_v, all_v, spm,
                      sem0, sem1):
        cid = lax.axis_index("c")
        sid = lax.axis_index("s")
        wid = sid * 2 + cid
        iota = _iota16()
        junkvec = jnp.full((_NL,), _JUNK, jnp.int32)

        def init_pst():
            for q in range(4):
                pst_v[pl.ds(q * _NL, _NL)] = junkvec

        init_pst()

        def vbase_of(w):
            return wid * _SHARD + w * _WIN

        def issue(w, buf, sem):
            vb = vbase_of(w)

            @pl.when(jnp.logical_and(w < _NWIN, vb + _WIN <= _VPAD))
            def _():
                pltpu.async_copy(tab_hbm.at[:, pl.ds(vb, _WIN)], buf, sem)

            @pl.when(jnp.logical_and(w < _NWIN,
                                     jnp.logical_and(vb < _VPAD,
                                                     vb + _WIN > _VPAD)))
            def _():
                pltpu.async_copy(tab_hbm.at[:, pl.ds(vb, 128)],
                                 buf.at[:, pl.ds(0, 128)], sem)

        def wait(w, buf, sem):
            vb = vbase_of(w)

            @pl.when(vb + _WIN <= _VPAD)
            def _():
                pltpu.make_async_copy(tab_hbm.at[:, pl.ds(0, _WIN)], buf,
                                      sem).wait()

            @pl.when(jnp.logical_and(vb < _VPAD, vb + _WIN > _VPAD))
            def _():
                pltpu.make_async_copy(tab_hbm.at[:, pl.ds(0, 128)],
                                      buf.at[:, pl.ds(0, 128)], sem).wait()

        fill_base = cid * _OCAP + sid * (_OCAP // 16)
        for f in range(_OCAP // 16 // 64):
            pltpu.sync_copy(
                pst_v,
                pos_hbm.at[pl.ds(pl.multiple_of(fill_base + f * 64, 64), 64)])
        plsc.subcore_barrier()

        issue(0, buf0, sem0)
        issue(1, buf1, sem1)

        pltpu.sync_copy(idx_hbm, idx_v)

        def filt(v, off):
            kv = idx_v[pl.ds(pl.multiple_of(v * _NL, _NL), _NL)]
            pv = _vfull(v * _NL) + iota
            m = (kv >> 15) == _vfull(wid)
            packed = ((kv & _vfull(_SHARD - 1)) << 15) | pv
            cs = plsc.cumsum(jnp.where(m, _vfull(1), _vfull(0)))
            tgt = jnp.where(m, _vfull(off - 1) + cs, _vfull(_SEG))
            plsc.store_scatter(pk_v, [tgt], packed)
            return off + cs[_NL - 1]

        n = lax.fori_loop(0, _B // _NL, filt, 0)
        nv = (n + _NL - 1) // _NL

        for q in range(4):
            plsc.store_scatter(meta_v, [_vfull(q * _NL) + iota], _vfull(0))

        def hist(j, carry):
            e = _vfull(j * _NL) + iota
            ev = plsc.load_gather(pk_v, [e])
            ones = jnp.where(e < _vfull(n), _vfull(1), _vfull(0))
            plsc.addupdate_scatter(meta_v, [ev >> 24], ones)
            return carry

        lax.fori_loop(0, nv, hist, 0)
        pad = 0
        for q in range(4):
            h = meta_v[pl.ds(q * _NL, _NL)]
            pq = jnp.where(h > _vfull(0),
                           (_vfull(16) - (h & _vfull(15))) & _vfull(15),
                           _vfull(0))
            pad = pad + plsc.cumsum(pq)[_NL - 1]
        np_ = n + pad

        out_len = (np_ + 63) & ~63
        _scalar_store(tmp_v, 0, out_len)
        pltpu.sync_copy(tmp_v,
                        spm.at[pl.ds(pl.multiple_of(sid * _NL, _NL), _NL)])
        plsc.subcore_barrier()
        pltpu.sync_copy(spm, all_v)
        lens = plsc.load_gather(all_v, [iota * _vfull(_NL)])
        excl = plsc.cumsum(lens) - lens
        plsc.store_scatter(tmp_v, [iota], excl)
        my_start = plsc.load_gather(tmp_v, [_vfull(sid)])[0]
        out_start = pl.multiple_of(cid * _OCAP + my_start, 64)

        def group(b, off2):
            def pick(j, o):
                e = _vfull(j * _NL) + iota
                ev = plsc.load_gather(pk_v, [e])
                m = (e < _vfull(n)) & ((ev >> 24) == _vfull(b))
                cs = plsc.cumsum(jnp.where(m, _vfull(1), _vfull(0)))
                tgt = jnp.where(m, _vfull(o - 1) + cs, _vfull(_SEG))
                plsc.store_scatter(pko_v, [tgt], ev)
                return o + cs[_NL - 1]

            end = lax.fori_loop(0, nv, pick, off2)
            sent = ((b * _WIN) << 15) | _JUNK
            plsc.store_scatter(pko_v, [_vfull(end) + iota], _vfull(sent))
            return (end + _NL - 1) & ~(_NL - 1)

        def extract(w, buf, g0, g1, carry):
            slot, optr = carry

            def chunk(j, carry2):
                slot, optr = carry2
                e = _vfull(g0 + j * _NL) + iota
                ev = plsc.load_gather(pko_v, [e])
                colv = (ev >> 15) - _vfull(w * _WIN)
                pvv = ev & _vfull(32767)

                def flush(slot, optr):
                    o = pl.multiple_of(optr, 64)
                    pltpu.sync_copy(st_v, out_hbm.at[pl.ds(o, 64)])
                    pltpu.sync_copy(pst_v, pos_hbm.at[pl.ds(o, 64)])
                    init_pst()
                    return 0, optr + 64

                for l in range(_NL):
                    col16 = _vfull(colv[l])
                    slot16 = _vfull(slot)
                    for q in range(4):
                        rows = _vfull(q * _NL) + iota
                        vals = plsc.load_gather(buf, [rows, col16])
                        plsc.store_scatter(st_v, [slot16, rows], vals)
                    plsc.store_scatter(pst_v, [slot16], _vfull(pvv[l]))
                    slot = slot + 1
                    slot, optr = lax.cond(slot == 64, flush,
                                          lambda s, o: (s, o), slot, optr)
                return slot, optr

            nvw = (g1 - g0) // _NL
            return lax.fori_loop(0, nvw, chunk, (slot, optr))

        def pair(g, carry):
            for p, (buf, sem) in enumerate(((buf0, sem0), (buf1, sem1))):
                start, slot, optr = carry
                w = 2 * g + p
                end = group(w, start)
                wait(w, buf, sem)
                slot, optr = extract(w, buf, start, end, (slot, optr))
                issue(w + 2, buf, sem)
                carry = (end, slot, optr)
            return carry

        _, slot, optr = lax.fori_loop(0, _NWIN // 2, pair,
                                      (0, 0, out_start))

        @pl.when(slot > 0)
        def _():
            o = pl.multiple_of(optr, 64)
            pltpu.sync_copy(st_v, out_hbm.at[pl.ds(o, 64)])
            pltpu.sync_copy(pst_v, pos_hbm.at[pl.ds(o, 64)])

    return gather_kernel(table_t, idx)


def _sc_unperm(out1, pos1):
    mesh = plsc.VectorSubcoreMesh(core_axis_name="c", subcore_axis_name="s")
    tot = 2 * _OCAP

    @functools.partial(
        pl.kernel,
        mesh=mesh,
        compiler_params=pltpu.CompilerParams(use_tc_tiling_on_sc=False,
                                             needs_layout_passes=False),
        out_type=jax.ShapeDtypeStruct((_B, _D), jnp.float32),
        scratch_types=[
            pltpu.VMEM((tot,), jnp.int32),
            pltpu.VMEM((5, 128), jnp.int32),
            pltpu.VMEM((512, _D), jnp.float32),
            pltpu.SemaphoreType.DMA,
        ],
    )
    def unperm_kernel(rows_hbm, pos_hbm, out_hbm, pos_v, inv_v,
                      rv, sem):
        cid = lax.axis_index("c")
        sid = lax.axis_index("s")
        wid = sid * 2 + cid
        base = wid * 512
        iota = _iota16()
        pltpu.sync_copy(pos_hbm, pos_v)

        def scan(v, carry):
            pv = pos_v[pl.ds(v * _NL, _NL)]
            loc = _vfull(v * _NL) + iota
            rel = pv - _vfull(base)
            m = (rel >= _vfull(0)) & (rel < _vfull(512))
            flat = jnp.where(m, rel, _vfull(512))
            plsc.store_scatter(inv_v, [flat >> 7, flat & _vfull(127)], loc)
            return carry

        lax.fori_loop(0, tot // _NL, scan, 0)
        for q in range(4):
            pltpu.async_copy(rows_hbm.at[inv_v.at[q]],
                             rv.at[pl.ds(q * 128, 128)], sem)
        pltpu.make_async_copy(rows_hbm.at[pl.ds(0, 512)], rv, sem).wait()
        pltpu.sync_copy(rv, out_hbm.at[pl.ds(pl.multiple_of(base, 128), 512)])

    return unperm_kernel(out1, pos1)


def _mlp_t_body(embt_ref, g_ref, i_ref, w1t_ref, b1_ref, w2t_ref, b2_ref,
                o_ref):
    blk = embt_ref.shape[1]
    e = embt_ref[...]
    g = g_ref[...]
    i = i_ref[...]
    row = lax.broadcasted_iota(jnp.int32, (_D, blk), 0)
    oht = ((row == g) | (row == i + _N_GARMENT)).astype(jnp.float32)
    cct = jnp.concatenate([e, oht], axis=0)
    ht = jnp.dot(w1t_ref[...], cct, preferred_element_type=jnp.float32)
    ht = jnp.maximum(ht + b1_ref[...], 0.0)
    ot = jnp.dot(w2t_ref[...], ht, preferred_element_type=jnp.float32)
    o_ref[...] = ot + b2_ref[...]


def _mlp_t(emb_t, gid, iid, w1pt, b1c, w2t, b2c):
    d, b = emb_t.shape
    blk = 2048
    grid = b // blk
    return pl.pallas_call(
        _mlp_t_body,
        grid=(grid,),
        in_specs=[
            pl.BlockSpec((d, blk), lambda i: (0, i)),
            pl.BlockSpec((1, blk), lambda i: (0, i)),
            pl.BlockSpec((1, blk), lambda i: (0, i)),
            pl.BlockSpec((d, 128), lambda i: (0, 0)),
            pl.BlockSpec((d, 1), lambda i: (0, 0)),
            pl.BlockSpec((d, d), lambda i: (0, 0)),
            pl.BlockSpec((d, 1), lambda i: (0, 0)),
        ],
        out_specs=pl.BlockSpec((d, blk), lambda i: (0, i)),
        out_shape=jax.ShapeDtypeStruct((d, b), jnp.float32),
    )(emb_t, gid.reshape(1, b), iid.reshape(1, b), w1pt, b1c, w2t, b2c)


def kernel(article_id, garment_group_name, index_group_name, emb_table,
           W1, b1, W2, b2):
    out1, pos1 = _sc_gather_packed(emb_table.T, article_id)
    emb = _sc_unperm(out1, pos1)
    d = W1.shape[1]
    w1pt = jnp.zeros((d, 128), W1.dtype).at[:, : W1.shape[0]].set(W1.T)
    out_t = _mlp_t(emb.T, garment_group_name, index_group_name, w1pt,
                   b1.reshape(d, 1), W2.T, b2.reshape(d, 1))
    return out_t.T
